# baseline (speedup 1.0000x reference)
import functools

import jax
import jax.numpy as jnp
from jax import lax
from jax.experimental import pallas as pl
from jax.experimental.pallas import tpu as pltpu
from jax.experimental.pallas import tpu_sc as plsc

B_BLK = 32
NBUF = 4
PACK = 8


def _sc_gather(emb128, x_hi):
    B = x_hi.shape[0]
    DP = emb128.shape[1]
    info = plsc.get_sparse_core_info()
    nw = info.num_cores * info.num_subcores
    b_per_w = B // nw

    mesh = plsc.VectorSubcoreMesh(core_axis_name="c", subcore_axis_name="s")

    @functools.partial(
        pl.kernel,
        mesh=mesh,
        out_type=jax.ShapeDtypeStruct((B, DP), jnp.float32),
        scratch_types=[
            pltpu.VMEM((b_per_w,), jnp.int32),
            pltpu.VMEM((b_per_w, DP), jnp.float32),
            pltpu.SemaphoreType.DMA,
        ],
    )
    def gather_kernel(emb_hbm, x_hbm, out_hbm, idx_v, rows_v, sem):
        wid = lax.axis_index("s") * info.num_cores + lax.axis_index("c")
        base = wid * b_per_w
        pltpu.sync_copy(x_hbm.at[pl.ds(base, b_per_w)], idx_v)
        pltpu.async_copy(emb_hbm.at[idx_v], rows_v, sem).wait()
        pltpu.sync_copy(rows_v, out_hbm.at[pl.ds(base, b_per_w)])

    return gather_kernel(emb128, x_hi)


def _mm_block(nsteps, e128_ref, off_ref, wt_ref, b_ref, out_hbm, bufs, sems):
    D = wt_ref.shape[0]
    jb = pl.program_id(0)
    slot = lax.rem(jb, NBUF)

    @pl.when(jb >= NBUF)
    def _reuse_wait():
        pltpu.make_async_copy(
            bufs.at[slot],
            out_hbm.at[pl.ds((jb - NBUF) * B_BLK, B_BLK), :],
            sems.at[slot],
        ).wait()

    off = off_ref[...]
    e = (off == 0.0) * e128_ref[:, 0:D]
    for k in range(1, PACK):
        e += (off == float(k)) * e128_ref[:, D * k : D * (k + 1)]
    bufs[slot] = (
        jnp.dot(
            e.astype(jnp.bfloat16),
            wt_ref[...],
            preferred_element_type=jnp.float32,
        )
        + b_ref[...]
    )

    pltpu.make_async_copy(
        bufs.at[slot],
        out_hbm.at[pl.ds(jb * B_BLK, B_BLK), :],
        sems.at[slot],
    ).start()

    @pl.when(jb == nsteps - 1)
    def _drain():
        for t in range(max(nsteps - NBUF, 0), nsteps):
            pltpu.make_async_copy(
                bufs.at[t % NBUF],
                out_hbm.at[pl.ds(t * B_BLK, B_BLK), :],
                sems.at[t % NBUF],
            ).wait()


def _transpose_block(w_ref, wt_ref):
    wt_ref[...] = w_ref[...].T.astype(jnp.bfloat16)


def _transpose_w(fc_w):
    N, D = fc_w.shape
    t_blk = 2048
    return pl.pallas_call(
        _transpose_block,
        grid=(pl.cdiv(N, t_blk),),
        in_specs=[pl.BlockSpec((t_blk, D), lambda j: (j, 0))],
        out_specs=pl.BlockSpec((D, t_blk), lambda j: (0, j)),
        out_shape=jax.ShapeDtypeStruct((D, N), jnp.bfloat16),
    )(fc_w)


def _tc_matmul(e128, off, wt, b2d):
    B, DP = e128.shape
    D, N = wt.shape
    nsteps = B // B_BLK
    return pl.pallas_call(
        functools.partial(_mm_block, nsteps),
        grid=(nsteps,),
        in_specs=[
            pl.BlockSpec((B_BLK, DP), lambda jb: (jb, 0)),
            pl.BlockSpec((B_BLK, 1), lambda jb: (jb, 0)),
            pl.BlockSpec((D, N), lambda jb: (0, 0)),
            pl.BlockSpec((1, N), lambda jb: (0, 0)),
        ],
        out_specs=pl.BlockSpec(memory_space=pltpu.MemorySpace.HBM),
        out_shape=jax.ShapeDtypeStruct((B, N), jnp.float32),
        scratch_shapes=[
            pltpu.VMEM((NBUF, B_BLK, N), jnp.float32),
            pltpu.SemaphoreType.DMA((NBUF,)),
        ],
        compiler_params=pltpu.CompilerParams(
            dimension_semantics=("arbitrary",)
        ),
    )(e128, off, wt, b2d)


def kernel(x, emb, fc_w, fc_b):
    V, D = emb.shape
    emb128 = emb.reshape(V // PACK, PACK * D)
    x_hi = (x >> 3).astype(jnp.int32)
    off = (x & 7).astype(jnp.float32).reshape(-1, 1)
    e128 = _sc_gather(emb128, x_hi)
    wt = _transpose_w(fc_w)
    return _tc_matmul(e128, off, wt, fc_b.reshape(1, -1))


# speedup vs baseline: 3.1885x; 3.1885x over previous
import functools

import jax
import jax.numpy as jnp
from jax import lax
from jax.experimental import pallas as pl
from jax.experimental.pallas import tpu as pltpu
from jax.experimental.pallas import tpu_sc as plsc

N_BLK = 2048
NBUF = 4
PACK = 8


def _sc_gather(emb128, x_hi):
    B = x_hi.shape[0]
    DP = emb128.shape[1]
    info = plsc.get_sparse_core_info()
    nw = info.num_cores * info.num_subcores
    b_per_w = B // nw

    mesh = plsc.VectorSubcoreMesh(core_axis_name="c", subcore_axis_name="s")

    @functools.partial(
        pl.kernel,
        mesh=mesh,
        out_type=jax.ShapeDtypeStruct((B, DP), jnp.float32),
        scratch_types=[
            pltpu.VMEM((b_per_w,), jnp.int32),
            pltpu.VMEM((b_per_w, DP), jnp.float32),
            pltpu.SemaphoreType.DMA,
        ],
    )
    def gather_kernel(emb_hbm, x_hbm, out_hbm, idx_v, rows_v, sem):
        wid = lax.axis_index("s") * info.num_cores + lax.axis_index("c")
        base = wid * b_per_w
        pltpu.sync_copy(x_hbm.at[pl.ds(base, b_per_w)], idx_v)
        pltpu.async_copy(emb_hbm.at[idx_v], rows_v, sem).wait()
        pltpu.sync_copy(rows_v, out_hbm.at[pl.ds(base, b_per_w)])

    return gather_kernel(emb128, x_hi)


def _mm_block(nsteps, rem, e128_ref, off2_ref, wtT_ref, b_ref, out_hbm,
              eT_ref, bufs, sems):
    D = wtT_ref.shape[0]
    j = pl.program_id(0)
    slot = lax.rem(j, NBUF)

    @pl.when(j == 0)
    def _select():
        e128T = e128_ref[...].T
        off = off2_ref[...]
        acc = (off == 0.0) * e128T[0:D, :]
        for k in range(1, PACK):
            acc += (off == float(k)) * e128T[D * k : D * (k + 1), :]
        eT_ref[...] = acc.astype(jnp.bfloat16)

    @pl.when(j >= NBUF)
    def _reuse_wait():
        pltpu.make_async_copy(
            bufs.at[slot],
            out_hbm.at[pl.ds((j - NBUF) * N_BLK, N_BLK), :],
            sems.at[slot],
        ).wait()

    bufs[slot] = (
        lax.dot_general(
            wtT_ref[...].astype(jnp.bfloat16),
            eT_ref[...],
            (((0,), (0,)), ((), ())),
            preferred_element_type=jnp.float32,
        )
        + b_ref[...].T
    )

    @pl.when(j < nsteps - 1)
    def _start_full():
        pltpu.make_async_copy(
            bufs.at[slot],
            out_hbm.at[pl.ds(j * N_BLK, N_BLK), :],
            sems.at[slot],
        ).start()

    @pl.when(j == nsteps - 1)
    def _start_last_and_drain():
        pltpu.make_async_copy(
            bufs.at[slot, :rem, :],
            out_hbm.at[pl.ds(j * N_BLK, rem), :],
            sems.at[slot],
        ).start()
        for t in range(max(nsteps - NBUF, 0), nsteps):
            w = rem if t == nsteps - 1 else N_BLK
            pltpu.make_async_copy(
                bufs.at[t % NBUF, :w, :],
                out_hbm.at[pl.ds(t * N_BLK, w), :],
                sems.at[t % NBUF],
            ).wait()


def _tc_matmul_t(e128, off2, wtT, b2d):
    B, DP = e128.shape
    D, N = wtT.shape
    nsteps = pl.cdiv(N, N_BLK)
    rem = N - (nsteps - 1) * N_BLK
    return pl.pallas_call(
        functools.partial(_mm_block, nsteps, rem),
        grid=(nsteps,),
        in_specs=[
            pl.BlockSpec((B, DP), lambda j: (0, 0)),
            pl.BlockSpec((1, B), lambda j: (0, 0)),
            pl.BlockSpec((D, N_BLK), lambda j: (0, j)),
            pl.BlockSpec((1, N_BLK), lambda j: (0, j)),
        ],
        out_specs=pl.BlockSpec(memory_space=pltpu.MemorySpace.HBM),
        out_shape=jax.ShapeDtypeStruct((N, B), jnp.float32),
        scratch_shapes=[
            pltpu.VMEM((D, B), jnp.bfloat16),
            pltpu.VMEM((NBUF, N_BLK, B), jnp.float32),
            pltpu.SemaphoreType.DMA((NBUF,)),
        ],
        compiler_params=pltpu.CompilerParams(
            dimension_semantics=("arbitrary",)
        ),
    )(e128, off2, wtT, b2d)


def kernel(x, emb, fc_w, fc_b):
    V, D = emb.shape
    emb128 = emb.reshape(V // PACK, PACK * D)
    x_hi = (x >> 3).astype(jnp.int32)
    off2 = (x & 7).astype(jnp.float32).reshape(1, -1)
    e128 = _sc_gather(emb128, x_hi)
    outT = _tc_matmul_t(e128, off2, fc_w.T, fc_b.reshape(1, -1))
    return outT.T
